# baseline (speedup 1.0000x reference)
import functools

import jax
import jax.numpy as jnp
from jax import lax
from jax.experimental import pallas as pl

_N = 5000
_NP = 5120
_BR = 256
_BC = 1280
_SIGMA = 0.5


def _nms_kernel(rows_ref, cols_ref, out_ref):
    rows = rows_ref[...]
    cx_r = rows[:, 0:1] * 100.0
    cy_r = rows[:, 1:2] * 100.0
    w_r = rows[:, 2:3] * 10.0 + 1e-3
    h_r = rows[:, 3:4] * 10.0 + 1e-3
    s_r = rows[:, 4:5]
    x1_r = cx_r - w_r * 0.5
    x2_r = cx_r + w_r * 0.5
    y1_r = cy_r - h_r * 0.5
    y2_r = cy_r + h_r * 0.5
    area_r = w_r * h_r

    def body(c, acc):
        cols = cols_ref[:, pl.ds(c * _BC, _BC)]
        cx_c = cols[0:1, :] * 100.0
        cy_c = cols[1:2, :] * 100.0
        w_c = cols[2:3, :] * 10.0 + 1e-3
        h_c = cols[3:4, :] * 10.0 + 1e-3
        s_c = cols[4:5, :]
        x1_c = cx_c - w_c * 0.5
        x2_c = cx_c + w_c * 0.5
        y1_c = cy_c - h_c * 0.5
        y2_c = cy_c + h_c * 0.5
        area_c = w_c * h_c
        iw = jnp.clip(jnp.minimum(x2_r, x2_c) - jnp.maximum(x1_r, x1_c), 0.0)
        ih = jnp.clip(jnp.minimum(y2_r, y2_c) - jnp.maximum(y1_r, y1_c), 0.0)
        inter = iw * ih
        iou = inter / (area_r + area_c - inter + 1e-8)
        hi = (s_c > s_r).astype(jnp.float32)
        return acc + jnp.sum(hi * iou * iou, axis=1, keepdims=True)

    acc = lax.fori_loop(0, _NP // _BC, body, jnp.zeros((_BR, 1), jnp.float32))
    new = s_r * jnp.exp(-acc / _SIGMA)
    out_ref[...] = jnp.where(new > 0.1, new, 0.0)


@jax.jit
def kernel(boxes, scores):
    feats = jnp.zeros((_NP, 8), jnp.float32)
    feats = feats.at[:_N, 0:4].set(boxes)
    feats = feats.at[:, 4].set(
        jnp.pad(scores, (0, _NP - _N), constant_values=-1.0))
    cols = feats.T

    out = pl.pallas_call(
        _nms_kernel,
        grid=(_NP // _BR,),
        in_specs=[
            pl.BlockSpec((_BR, 8), lambda r: (r, 0)),
            pl.BlockSpec((8, _NP), lambda r: (0, 0)),
        ],
        out_specs=pl.BlockSpec((_BR, 1), lambda r: (r, 0)),
        out_shape=jax.ShapeDtypeStruct((_NP, 1), jnp.float32),
    )(feats, cols)
    return out[:_N, 0]
